# baseline (speedup 1.0000x reference)
import functools

import jax
import jax.numpy as jnp
from jax import lax
from jax.experimental import pallas as pl
from jax.experimental.pallas import tpu as pltpu
from jax.experimental.pallas import tpu_sc as plsc

BLK = 512
MBLK = 2048
CH = 32


def _nn_body(x1t_ref, x2tn_ref, gidx_ref, wts_ref):
    n2 = x2tn_ref.shape[2]
    x1t = x1t_ref[0]
    x2tn = x2tn_ref[0]

    d0 = x1t[0, :].reshape(BLK, 1) + x2tn[0, :].reshape(1, n2)
    d1 = x1t[1, :].reshape(BLK, 1) + x2tn[1, :].reshape(1, n2)
    dd = x1t[2, :].reshape(BLK, 1) + x2tn[2, :].reshape(1, n2)
    d2 = d0 * d0 + d1 * d1 + dd * dd

    colf = jax.lax.broadcasted_iota(
        jnp.int32, (BLK, n2), 1).astype(jnp.float32)
    big = jnp.float32(jnp.inf)
    nf = jnp.float32(n2)

    def rowmin(x):
        return jnp.min(x, axis=1)

    m1v = rowmin(d2)
    m1 = m1v.reshape(BLK, 1)
    a1v = rowmin(jnp.where(d2 == m1, colf, nf))
    a1 = a1v.reshape(BLK, 1)
    t2 = jnp.where(colf == a1, big, d2)
    m2v = rowmin(t2)
    m2 = m2v.reshape(BLK, 1)
    a2v = rowmin(jnp.where(t2 == m2, colf, nf))
    a2 = a2v.reshape(BLK, 1)
    t3 = jnp.where(colf == a2, big, t2)
    m3v = rowmin(t3)
    m3 = m3v.reshape(BLK, 1)
    a3v = rowmin(jnp.where(t3 == m3, colf, nf))

    r1 = 1.0 / jnp.maximum(m1v, jnp.float32(1e-8))
    r2 = 1.0 / jnp.maximum(m2v, jnp.float32(1e-8))
    r3 = 1.0 / jnp.maximum(m3v, jnp.float32(1e-8))
    inv_norm = 1.0 / (r1 + r2 + r3)

    b = pl.program_id(0)
    base = b * n2
    idx3 = jnp.concatenate(
        [a1v.reshape(1, BLK), a2v.reshape(1, BLK), a3v.reshape(1, BLK)],
        axis=0).astype(jnp.int32) + base
    wts3 = jnp.concatenate(
        [(r1 * inv_norm).reshape(1, BLK), (r2 * inv_norm).reshape(1, BLK),
         (r3 * inv_norm).reshape(1, BLK)], axis=0)
    gidx_ref[0] = idx3
    wts_ref[0] = wts3


def _three_nn_tc(x1t, x2tn):
    B, _, n1 = x1t.shape
    n2 = x2tn.shape[2]
    grid = (B, n1 // BLK)
    return pl.pallas_call(
        _nn_body,
        grid=grid,
        in_specs=[
            pl.BlockSpec((1, 3, BLK), lambda b, i: (b, 0, i)),
            pl.BlockSpec((1, 3, n2), lambda b, i: (b, 0, 0)),
        ],
        out_specs=[
            pl.BlockSpec((1, 3, BLK), lambda b, i: (b, 0, i)),
            pl.BlockSpec((1, 3, BLK), lambda b, i: (b, 0, i)),
        ],
        out_shape=[
            jax.ShapeDtypeStruct((B, 3, n1), jnp.int32),
            jax.ShapeDtypeStruct((B, 3, n1), jnp.float32),
        ],
        compiler_params=pltpu.CompilerParams(
            dimension_semantics=("parallel", "parallel"),
        ),
    )(x1t, x2tn)


def _make_sc_gather(B, n1, n2, c2):
    info = plsc.get_sparse_core_info()
    nc, ns = info.num_cores, info.num_subcores
    nw = nc * ns
    rpw = (B * n1) // nw
    wpb = n1 // rpw
    nch = rpw // CH
    mesh = plsc.VectorSubcoreMesh(core_axis_name="c", subcore_axis_name="s")

    @functools.partial(
        pl.kernel,
        out_type=jax.ShapeDtypeStruct((B, n1, c2), jnp.float32),
        mesh=mesh,
        scratch_types=[
            pltpu.VMEM((rpw,), jnp.int32),
            pltpu.VMEM((rpw,), jnp.int32),
            pltpu.VMEM((rpw,), jnp.int32),
            pltpu.VMEM((rpw,), jnp.float32),
            pltpu.VMEM((rpw,), jnp.float32),
            pltpu.VMEM((rpw,), jnp.float32),
            pltpu.VMEM((CH, c2), jnp.float32),
            pltpu.VMEM((CH, c2), jnp.float32),
            pltpu.VMEM((CH, c2), jnp.float32),
            pltpu.VMEM((CH, c2), jnp.float32),
            pltpu.VMEM((CH, c2), jnp.float32),
            pltpu.VMEM((CH, c2), jnp.float32),
            pltpu.VMEM((CH, c2), jnp.float32),
            pltpu.SemaphoreType.DMA,
            pltpu.SemaphoreType.DMA,
        ],
    )
    def sc_gather(p2_hbm, gidx_hbm, wts_hbm, out_hbm,
                  ia1, ia2, ia3, wa1, wa2, wa3,
                  ga1, ga2, ga3, gb1, gb2, gb3, acc,
                  sem_a, sem_b):
        wid = lax.axis_index("s") * nc + lax.axis_index("c")
        b = wid // wpb
        i0 = (wid % wpb) * rpw

        idx_bufs = (ia1, ia2, ia3)
        wt_bufs = (wa1, wa2, wa3)
        for k in range(3):
            off = pl.multiple_of((b * 3 + k) * n1 + i0, 8)
            pltpu.sync_copy(gidx_hbm.at[pl.ds(off, rpw)], idx_bufs[k])
            pltpu.sync_copy(wts_hbm.at[pl.ds(off, rpw)], wt_bufs[k])

        def fire(ch, bufs, sem):
            off = jnp.minimum(ch, nch - 1) * CH
            for k, g in enumerate(bufs):
                pltpu.async_copy(p2_hbm.at[idx_bufs[k].at[pl.ds(off, CH)]],
                                 g, sem)

        def drain(bufs, sem):
            for g in bufs:
                pltpu.make_async_copy(p2_hbm.at[pl.ds(0, CH)], g, sem).wait()

        def consume(ch, bufs):
            g1, g2, g3 = bufs

            def grp_body(j, _):
                goff = ch * CH + j * 16
                wv1 = wa1[pl.ds(goff, 16)]
                wv2 = wa2[pl.ds(goff, 16)]
                wv3 = wa3[pl.ds(goff, 16)]
                for r in range(16):
                    i = j * 16 + r
                    s1, s2, s3 = wv1[r], wv2[r], wv3[r]
                    for f in range(c2 // 16):
                        sl = pl.ds(f * 16, 16)
                        acc[i, sl] = (s1 * g1[i, sl] + s2 * g2[i, sl]
                                      + s3 * g3[i, sl])
                return _

            lax.fori_loop(0, CH // 16, grp_body, None)
            pltpu.sync_copy(acc, out_hbm.at[b, pl.ds(i0 + ch * CH, CH), :])

        bufs_a = (ga1, ga2, ga3)
        bufs_b = (gb1, gb2, gb3)
        fire(jnp.int32(0), bufs_a, sem_a)
        fire(jnp.int32(1), bufs_b, sem_b)

        def chunk_pair(t, _):
            ch_a = 2 * t
            drain(bufs_a, sem_a)
            consume(ch_a, bufs_a)
            fire(ch_a + 2, bufs_a, sem_a)
            ch_b = 2 * t + 1
            drain(bufs_b, sem_b)
            consume(ch_b, bufs_b)
            fire(ch_b + 2, bufs_b, sem_b)
            return _

        lax.fori_loop(0, nch // 2, chunk_pair, None)
        drain(bufs_a, sem_a)
        drain(bufs_b, sem_b)

    return sc_gather


def _fused_body(x1t_ref, x2tn_ref, p1_ref, p2_ref, w1t_ref, b1_ref,
                w2t_ref, b2_ref, out_ref):
    n2 = x2tn_ref.shape[2]
    x1t = x1t_ref[0]
    x2tn = x2tn_ref[0]

    d0 = x1t[0, :].reshape(BLK, 1) + x2tn[0, :].reshape(1, n2)
    d1 = x1t[1, :].reshape(BLK, 1) + x2tn[1, :].reshape(1, n2)
    dd = x1t[2, :].reshape(BLK, 1) + x2tn[2, :].reshape(1, n2)
    d2 = d0 * d0 + d1 * d1 + dd * dd

    colf = jax.lax.broadcasted_iota(
        jnp.int32, (BLK, n2), 1).astype(jnp.float32)
    big = jnp.float32(jnp.inf)
    nf = jnp.float32(n2)

    def rowmin(x):
        return jnp.min(x, axis=1).reshape(BLK, 1)

    m1 = rowmin(d2)
    a1 = rowmin(jnp.where(d2 == m1, colf, nf))
    c1 = colf == a1
    t2 = jnp.where(c1, big, d2)
    m2 = rowmin(t2)
    a2 = rowmin(jnp.where(t2 == m2, colf, nf))
    c2m = colf == a2
    t3 = jnp.where(c2m, big, t2)
    m3 = rowmin(t3)
    a3 = rowmin(jnp.where(t3 == m3, colf, nf))
    c3 = colf == a3

    r1 = 1.0 / jnp.maximum(m1, jnp.float32(1e-8))
    r2 = 1.0 / jnp.maximum(m2, jnp.float32(1e-8))
    r3 = 1.0 / jnp.maximum(m3, jnp.float32(1e-8))
    inv_norm = 1.0 / (r1 + r2 + r3)
    zero = jnp.float32(0.0)
    s = jnp.where(c1, r1 * inv_norm,
                  jnp.where(c2m, r2 * inv_norm,
                            jnp.where(c3, r3 * inv_norm, zero)))

    interp = jnp.dot(s, p2_ref[0], preferred_element_type=jnp.float32)

    w1t = w1t_ref[...]
    cc2 = interp.shape[1]
    h = jnp.dot(interp, w1t[:cc2, :], preferred_element_type=jnp.float32)
    h = h + jnp.dot(p1_ref[0], w1t[cc2:, :],
                    preferred_element_type=jnp.float32)
    h = jnp.maximum(h + b1_ref[...], 0.0)
    o = jnp.dot(h, w2t_ref[...], preferred_element_type=jnp.float32)
    o = jnp.maximum(o + b2_ref[...], 0.0)
    out_ref[0] = o


def _fused_tc(x1t, x2tn, points1, points2, w1t, b1r, w2t, b2r):
    B, _, n1 = x1t.shape
    n2 = x2tn.shape[2]
    c1 = points1.shape[2]
    c2 = points2.shape[2]
    cout = w2t.shape[1]
    grid = (B, n1 // BLK)
    return pl.pallas_call(
        _fused_body,
        grid=grid,
        in_specs=[
            pl.BlockSpec((1, 3, BLK), lambda b, i: (b, 0, i)),
            pl.BlockSpec((1, 3, n2), lambda b, i: (b, 0, 0)),
            pl.BlockSpec((1, BLK, c1), lambda b, i: (b, i, 0)),
            pl.BlockSpec((1, n2, c2), lambda b, i: (b, 0, 0)),
            pl.BlockSpec((c2 + c1, cout), lambda b, i: (0, 0)),
            pl.BlockSpec((1, cout), lambda b, i: (0, 0)),
            pl.BlockSpec((cout, cout), lambda b, i: (0, 0)),
            pl.BlockSpec((1, cout), lambda b, i: (0, 0)),
        ],
        out_specs=pl.BlockSpec((1, BLK, cout), lambda b, i: (b, i, 0)),
        out_shape=jax.ShapeDtypeStruct((B, n1, cout), jnp.float32),
        compiler_params=pltpu.CompilerParams(
            dimension_semantics=("parallel", "parallel"),
        ),
    )(x1t, x2tn, points1, points2, w1t, b1r, w2t, b2r)


def _mlp_body(interp_ref, p1_ref, w1t_ref, b1_ref, w2t_ref, b2_ref, out_ref):
    interp = interp_ref[0]
    w1t = w1t_ref[...]
    c2 = interp.shape[1]
    h = jnp.dot(interp, w1t[:c2, :], preferred_element_type=jnp.float32)
    h = h + jnp.dot(p1_ref[0], w1t[c2:, :], preferred_element_type=jnp.float32)
    h = jnp.maximum(h + b1_ref[...], 0.0)
    o = jnp.dot(h, w2t_ref[...], preferred_element_type=jnp.float32)
    o = jnp.maximum(o + b2_ref[...], 0.0)
    out_ref[0] = o


def _mlp_tc(interp, points1, w1t, b1r, w2t, b2r):
    B, n1, c2 = interp.shape
    c1 = points1.shape[2]
    cout = w2t.shape[1]
    grid = (B, n1 // MBLK)
    return pl.pallas_call(
        _mlp_body,
        grid=grid,
        in_specs=[
            pl.BlockSpec((1, MBLK, c2), lambda b, i: (b, i, 0)),
            pl.BlockSpec((1, MBLK, c1), lambda b, i: (b, i, 0)),
            pl.BlockSpec((c2 + c1, cout), lambda b, i: (0, 0)),
            pl.BlockSpec((1, cout), lambda b, i: (0, 0)),
            pl.BlockSpec((cout, cout), lambda b, i: (0, 0)),
            pl.BlockSpec((1, cout), lambda b, i: (0, 0)),
        ],
        out_specs=pl.BlockSpec((1, MBLK, cout), lambda b, i: (b, i, 0)),
        out_shape=jax.ShapeDtypeStruct((B, n1, cout), jnp.float32),
        compiler_params=pltpu.CompilerParams(
            dimension_semantics=("parallel", "parallel"),
        ),
    )(interp, points1, w1t, b1r, w2t, b2r)


@jax.jit
def kernel(xyz1, xyz2, points1, points2, W1, b1, W2, b2):
    B, n1, _ = xyz1.shape
    n2 = xyz2.shape[1]
    c2 = points2.shape[2]
    SPLIT = 2

    x1t = jnp.swapaxes(xyz1, 1, 2)
    x2tn = -jnp.swapaxes(xyz2, 1, 2)
    w1t = W1.T
    w2t = W2.T
    b1r = b1.reshape(1, -1)
    b2r = b2.reshape(1, -1)

    gidx, wts = _three_nn_tc(x1t[:SPLIT], x2tn[:SPLIT])
    p2flat = points2[:SPLIT].reshape(SPLIT * n2, c2)
    interp = _make_sc_gather(SPLIT, n1, n2, c2)(
        p2flat, gidx.reshape(SPLIT * 3 * n1), wts.reshape(SPLIT * 3 * n1))

    out_tc = _fused_tc(x1t[SPLIT:], x2tn[SPLIT:], points1[SPLIT:],
                       points2[SPLIT:], w1t, b1r, w2t, b2r)

    out_sc = _mlp_tc(interp, points1[:SPLIT], w1t, b1r, w2t, b2r)
    return jnp.concatenate([out_sc, out_tc], axis=0)


# speedup vs baseline: 1.0569x; 1.0569x over previous
import functools

import jax
import jax.numpy as jnp
from jax import lax
from jax.experimental import pallas as pl
from jax.experimental.pallas import tpu as pltpu
from jax.experimental.pallas import tpu_sc as plsc

BLK = 512
MBLK = 2048
CH = 32


def _nn_body(x1t_ref, x2tn_ref, gidx_ref, wts_ref):
    n2 = x2tn_ref.shape[2]
    x1t = x1t_ref[0]
    x2tn = x2tn_ref[0]

    d0 = x1t[0, :].reshape(BLK, 1) + x2tn[0, :].reshape(1, n2)
    d1 = x1t[1, :].reshape(BLK, 1) + x2tn[1, :].reshape(1, n2)
    dd = x1t[2, :].reshape(BLK, 1) + x2tn[2, :].reshape(1, n2)
    d2 = d0 * d0 + d1 * d1 + dd * dd

    colf = jax.lax.broadcasted_iota(
        jnp.int32, (BLK, n2), 1).astype(jnp.float32)
    big = jnp.float32(jnp.inf)
    nf = jnp.float32(n2)

    def rowmin(x):
        return jnp.min(x, axis=1)

    m1v = rowmin(d2)
    m1 = m1v.reshape(BLK, 1)
    a1v = rowmin(jnp.where(d2 == m1, colf, nf))
    a1 = a1v.reshape(BLK, 1)
    t2 = jnp.where(colf == a1, big, d2)
    m2v = rowmin(t2)
    m2 = m2v.reshape(BLK, 1)
    a2v = rowmin(jnp.where(t2 == m2, colf, nf))
    a2 = a2v.reshape(BLK, 1)
    t3 = jnp.where(colf == a2, big, t2)
    m3v = rowmin(t3)
    m3 = m3v.reshape(BLK, 1)
    a3v = rowmin(jnp.where(t3 == m3, colf, nf))

    r1 = 1.0 / jnp.maximum(m1v, jnp.float32(1e-8))
    r2 = 1.0 / jnp.maximum(m2v, jnp.float32(1e-8))
    r3 = 1.0 / jnp.maximum(m3v, jnp.float32(1e-8))
    inv_norm = 1.0 / (r1 + r2 + r3)

    b = pl.program_id(0)
    base = b * n2
    idx3 = jnp.concatenate(
        [a1v.reshape(1, BLK), a2v.reshape(1, BLK), a3v.reshape(1, BLK)],
        axis=0).astype(jnp.int32) + base
    wts3 = jnp.concatenate(
        [(r1 * inv_norm).reshape(1, BLK), (r2 * inv_norm).reshape(1, BLK),
         (r3 * inv_norm).reshape(1, BLK)], axis=0)
    gidx_ref[0] = idx3
    wts_ref[0] = wts3


def _three_nn_tc(x1t, x2tn):
    B, _, n1 = x1t.shape
    n2 = x2tn.shape[2]
    grid = (B, n1 // BLK)
    return pl.pallas_call(
        _nn_body,
        grid=grid,
        in_specs=[
            pl.BlockSpec((1, 3, BLK), lambda b, i: (b, 0, i)),
            pl.BlockSpec((1, 3, n2), lambda b, i: (b, 0, 0)),
        ],
        out_specs=[
            pl.BlockSpec((1, 3, BLK), lambda b, i: (b, 0, i)),
            pl.BlockSpec((1, 3, BLK), lambda b, i: (b, 0, i)),
        ],
        out_shape=[
            jax.ShapeDtypeStruct((B, 3, n1), jnp.int32),
            jax.ShapeDtypeStruct((B, 3, n1), jnp.float32),
        ],
        compiler_params=pltpu.CompilerParams(
            dimension_semantics=("parallel", "parallel"),
        ),
    )(x1t, x2tn)


def _make_sc_gather(B, n1, n2, c2):
    info = plsc.get_sparse_core_info()
    nc, ns = info.num_cores, info.num_subcores
    nw = nc * ns
    rpw = (B * n1) // nw
    wpb = n1 // rpw
    nch = rpw // CH
    mesh = plsc.VectorSubcoreMesh(core_axis_name="c", subcore_axis_name="s")

    @functools.partial(
        pl.kernel,
        out_type=jax.ShapeDtypeStruct((B, n1, c2), jnp.float32),
        mesh=mesh,
        scratch_types=[
            pltpu.VMEM((rpw,), jnp.int32),
            pltpu.VMEM((rpw,), jnp.int32),
            pltpu.VMEM((rpw,), jnp.int32),
            pltpu.VMEM((rpw,), jnp.float32),
            pltpu.VMEM((rpw,), jnp.float32),
            pltpu.VMEM((rpw,), jnp.float32),
            pltpu.VMEM((CH, c2), jnp.float32),
            pltpu.VMEM((CH, c2), jnp.float32),
            pltpu.VMEM((CH, c2), jnp.float32),
            pltpu.VMEM((CH, c2), jnp.float32),
            pltpu.VMEM((CH, c2), jnp.float32),
            pltpu.VMEM((CH, c2), jnp.float32),
            pltpu.VMEM((CH, c2), jnp.float32),
            pltpu.SemaphoreType.DMA,
            pltpu.SemaphoreType.DMA,
        ],
    )
    def sc_gather(p2_hbm, gidx_hbm, wts_hbm, out_hbm,
                  ia1, ia2, ia3, wa1, wa2, wa3,
                  ga1, ga2, ga3, gb1, gb2, gb3, acc,
                  sem_a, sem_b):
        wid = lax.axis_index("s") * nc + lax.axis_index("c")
        b = wid // wpb
        i0 = (wid % wpb) * rpw

        idx_bufs = (ia1, ia2, ia3)
        wt_bufs = (wa1, wa2, wa3)
        for k in range(3):
            off = pl.multiple_of((b * 3 + k) * n1 + i0, 8)
            pltpu.sync_copy(gidx_hbm.at[pl.ds(off, rpw)], idx_bufs[k])
            pltpu.sync_copy(wts_hbm.at[pl.ds(off, rpw)], wt_bufs[k])

        def fire(ch, bufs, sem):
            off = jnp.minimum(ch, nch - 1) * CH
            for k, g in enumerate(bufs):
                pltpu.async_copy(p2_hbm.at[idx_bufs[k].at[pl.ds(off, CH)]],
                                 g, sem)

        def drain(bufs, sem):
            for g in bufs:
                pltpu.make_async_copy(p2_hbm.at[pl.ds(0, CH)], g, sem).wait()

        def consume(ch, bufs):
            g1, g2, g3 = bufs

            def grp_body(j, _):
                goff = ch * CH + j * 16
                wv1 = wa1[pl.ds(goff, 16)]
                wv2 = wa2[pl.ds(goff, 16)]
                wv3 = wa3[pl.ds(goff, 16)]
                for r in range(16):
                    i = j * 16 + r
                    s1, s2, s3 = wv1[r], wv2[r], wv3[r]
                    for f in range(c2 // 16):
                        sl = pl.ds(f * 16, 16)
                        acc[i, sl] = (s1 * g1[i, sl] + s2 * g2[i, sl]
                                      + s3 * g3[i, sl])
                return _

            lax.fori_loop(0, CH // 16, grp_body, None)
            pltpu.sync_copy(acc, out_hbm.at[b, pl.ds(i0 + ch * CH, CH), :])

        bufs_a = (ga1, ga2, ga3)
        bufs_b = (gb1, gb2, gb3)
        fire(jnp.int32(0), bufs_a, sem_a)
        fire(jnp.int32(1), bufs_b, sem_b)

        def chunk_pair(t, _):
            ch_a = 2 * t
            drain(bufs_a, sem_a)
            consume(ch_a, bufs_a)
            fire(ch_a + 2, bufs_a, sem_a)
            ch_b = 2 * t + 1
            drain(bufs_b, sem_b)
            consume(ch_b, bufs_b)
            fire(ch_b + 2, bufs_b, sem_b)
            return _

        lax.fori_loop(0, nch // 2, chunk_pair, None)
        drain(bufs_a, sem_a)
        drain(bufs_b, sem_b)

    return sc_gather


def _fused_body(x1t_ref, x2tn_ref, p1_ref, p2_ref, w1t_ref, b1_ref,
                w2t_ref, b2_ref, out_ref):
    n2 = x2tn_ref.shape[2]
    x1t = x1t_ref[0]
    x2tn = x2tn_ref[0]

    d0 = x1t[0, :].reshape(BLK, 1) + x2tn[0, :].reshape(1, n2)
    d1 = x1t[1, :].reshape(BLK, 1) + x2tn[1, :].reshape(1, n2)
    dd = x1t[2, :].reshape(BLK, 1) + x2tn[2, :].reshape(1, n2)
    d2 = d0 * d0 + d1 * d1 + dd * dd

    colf = jax.lax.broadcasted_iota(
        jnp.int32, (BLK, n2), 1).astype(jnp.float32)
    big = jnp.float32(jnp.inf)
    nf = jnp.float32(n2)

    def rowmin(x):
        return jnp.min(x, axis=1).reshape(BLK, 1)

    m1 = rowmin(d2)
    a1 = rowmin(jnp.where(d2 == m1, colf, nf))
    c1 = colf == a1
    t2 = jnp.where(c1, big, d2)
    m2 = rowmin(t2)
    a2 = rowmin(jnp.where(t2 == m2, colf, nf))
    c2m = colf == a2
    t3 = jnp.where(c2m, big, t2)
    m3 = rowmin(t3)
    a3 = rowmin(jnp.where(t3 == m3, colf, nf))
    c3 = colf == a3

    r1 = 1.0 / jnp.maximum(m1, jnp.float32(1e-8))
    r2 = 1.0 / jnp.maximum(m2, jnp.float32(1e-8))
    r3 = 1.0 / jnp.maximum(m3, jnp.float32(1e-8))
    inv_norm = 1.0 / (r1 + r2 + r3)
    zero = jnp.float32(0.0)
    s = jnp.where(c1, r1 * inv_norm,
                  jnp.where(c2m, r2 * inv_norm,
                            jnp.where(c3, r3 * inv_norm, zero)))

    interp = jnp.dot(s, p2_ref[0], preferred_element_type=jnp.float32)

    w1t = w1t_ref[...]
    cc2 = interp.shape[1]
    h = jnp.dot(interp, w1t[:cc2, :], preferred_element_type=jnp.float32)
    h = h + jnp.dot(p1_ref[0], w1t[cc2:, :],
                    preferred_element_type=jnp.float32)
    h = jnp.maximum(h + b1_ref[...], 0.0)
    o = jnp.dot(h, w2t_ref[...], preferred_element_type=jnp.float32)
    o = jnp.maximum(o + b2_ref[...], 0.0)
    out_ref[0] = o


def _fused_tc(x1t, x2tn, points1, points2, w1t, b1r, w2t, b2r):
    B, _, n1 = x1t.shape
    n2 = x2tn.shape[2]
    c1 = points1.shape[2]
    c2 = points2.shape[2]
    cout = w2t.shape[1]
    grid = (B, n1 // BLK)
    return pl.pallas_call(
        _fused_body,
        grid=grid,
        in_specs=[
            pl.BlockSpec((1, 3, BLK), lambda b, i: (b, 0, i)),
            pl.BlockSpec((1, 3, n2), lambda b, i: (b, 0, 0)),
            pl.BlockSpec((1, BLK, c1), lambda b, i: (b, i, 0)),
            pl.BlockSpec((1, n2, c2), lambda b, i: (b, 0, 0)),
            pl.BlockSpec((c2 + c1, cout), lambda b, i: (0, 0)),
            pl.BlockSpec((1, cout), lambda b, i: (0, 0)),
            pl.BlockSpec((cout, cout), lambda b, i: (0, 0)),
            pl.BlockSpec((1, cout), lambda b, i: (0, 0)),
        ],
        out_specs=pl.BlockSpec((1, BLK, cout), lambda b, i: (b, i, 0)),
        out_shape=jax.ShapeDtypeStruct((B, n1, cout), jnp.float32),
        compiler_params=pltpu.CompilerParams(
            dimension_semantics=("parallel", "parallel"),
        ),
    )(x1t, x2tn, points1, points2, w1t, b1r, w2t, b2r)


def _mlp_body(interp_ref, p1_ref, w1t_ref, b1_ref, w2t_ref, b2_ref, out_ref):
    interp = interp_ref[0]
    w1t = w1t_ref[...]
    c2 = interp.shape[1]
    h = jnp.dot(interp, w1t[:c2, :], preferred_element_type=jnp.float32)
    h = h + jnp.dot(p1_ref[0], w1t[c2:, :], preferred_element_type=jnp.float32)
    h = jnp.maximum(h + b1_ref[...], 0.0)
    o = jnp.dot(h, w2t_ref[...], preferred_element_type=jnp.float32)
    o = jnp.maximum(o + b2_ref[...], 0.0)
    out_ref[0] = o


def _mlp_tc(interp, points1, w1t, b1r, w2t, b2r):
    B, n1, c2 = interp.shape
    c1 = points1.shape[2]
    cout = w2t.shape[1]
    grid = (B, n1 // MBLK)
    return pl.pallas_call(
        _mlp_body,
        grid=grid,
        in_specs=[
            pl.BlockSpec((1, MBLK, c2), lambda b, i: (b, i, 0)),
            pl.BlockSpec((1, MBLK, c1), lambda b, i: (b, i, 0)),
            pl.BlockSpec((c2 + c1, cout), lambda b, i: (0, 0)),
            pl.BlockSpec((1, cout), lambda b, i: (0, 0)),
            pl.BlockSpec((cout, cout), lambda b, i: (0, 0)),
            pl.BlockSpec((1, cout), lambda b, i: (0, 0)),
        ],
        out_specs=pl.BlockSpec((1, MBLK, cout), lambda b, i: (b, i, 0)),
        out_shape=jax.ShapeDtypeStruct((B, n1, cout), jnp.float32),
        compiler_params=pltpu.CompilerParams(
            dimension_semantics=("parallel", "parallel"),
        ),
    )(interp, points1, w1t, b1r, w2t, b2r)


@jax.jit
def kernel(xyz1, xyz2, points1, points2, W1, b1, W2, b2):
    B, n1, _ = xyz1.shape
    n2 = xyz2.shape[1]
    c2 = points2.shape[2]
    SPLIT = 4

    x1t = jnp.swapaxes(xyz1, 1, 2)
    x2tn = -jnp.swapaxes(xyz2, 1, 2)
    w1t = W1.T
    w2t = W2.T
    b1r = b1.reshape(1, -1)
    b2r = b2.reshape(1, -1)

    gidx, wts = _three_nn_tc(x1t[:SPLIT], x2tn[:SPLIT])
    p2flat = points2[:SPLIT].reshape(SPLIT * n2, c2)
    interp = _make_sc_gather(SPLIT, n1, n2, c2)(
        p2flat, gidx.reshape(SPLIT * 3 * n1), wts.reshape(SPLIT * 3 * n1))

    out_tc = _fused_tc(x1t[SPLIT:], x2tn[SPLIT:], points1[SPLIT:],
                       points2[SPLIT:], w1t, b1r, w2t, b2r)

    out_sc = _mlp_tc(interp, points1[:SPLIT], w1t, b1r, w2t, b2r)
    return jnp.concatenate([out_sc, out_tc], axis=0)
